# Initial kernel scaffold; baseline (speedup 1.0000x reference)
import jax
import jax.numpy as jnp
from jax.experimental import pallas as pl


def kernel(x, edge_index, W1l, W1r, b1, W2l, W2r, b2):
    raise NotImplementedError("write your pallas kernel here")


# speedup vs baseline: 2.6784x; 2.6784x over previous
import functools

import jax
import jax.numpy as jnp
from jax import lax
from jax.experimental import pallas as pl
from jax.experimental.pallas import tpu as pltpu
from jax.experimental.pallas import tpu_sc as plsc

N = 10000
E = 160000
D = 256
H = 128
N_PAD = 10112
E_PAD = 163840
CH = 128
CHUNKS = E_PAD // (16 * CH)
ROWS_Z = N_PAD // 16
ROWS_W = 632
ROWS_W_LAST = N - 15 * ROWS_W
RB = 400
GRID = N // RB

_f32 = jnp.float32


def _dot_t(a, w):
    return lax.dot_general(a, w, (((1,), (1,)), ((), ())),
                           preferred_element_type=_f32)


def _mm1_body(x_ref, w1l_ref, w1r_ref, b1_ref, y0_ref, y1_ref, z1_ref):
    xb = x_ref[...]
    yl = _dot_t(xb, w1l_ref[...])
    y0_ref[...] = yl[:, :H]
    y1_ref[...] = yl[:, H:]
    z1_ref[...] = _dot_t(xb, w1r_ref[...]) + b1_ref[...]


def _mm1(x, w1l, w1r, b1r):
    return pl.pallas_call(
        _mm1_body,
        grid=(GRID,),
        in_specs=[
            pl.BlockSpec((RB, D), lambda i: (i, 0)),
            pl.BlockSpec((D, D), lambda i: (0, 0)),
            pl.BlockSpec((D, D), lambda i: (0, 0)),
            pl.BlockSpec((1, D), lambda i: (0, 0)),
        ],
        out_specs=[
            pl.BlockSpec((RB, H), lambda i: (i, 0)),
            pl.BlockSpec((RB, H), lambda i: (i, 0)),
            pl.BlockSpec((RB, D), lambda i: (i, 0)),
        ],
        out_shape=[
            jax.ShapeDtypeStruct((N, H), _f32),
            jax.ShapeDtypeStruct((N, H), _f32),
            jax.ShapeDtypeStruct((N, D), _f32),
        ],
    )(x, w1l, w1r, b1r)


def _mm2_body(a0_ref, a1_ref, z1_ref, deg_ref, w2l_ref, w2r_ref, b2_ref,
              y0_ref, y1_ref, z2_ref):
    dinv = 1.0 / jnp.maximum(deg_ref[...], 1.0)
    z1 = z1_ref[...]
    h0 = jnp.maximum(a0_ref[...] * dinv + z1[:, :H], 0.0)
    h1 = jnp.maximum(a1_ref[...] * dinv + z1[:, H:], 0.0)
    w2l = w2l_ref[...]
    w2r = w2r_ref[...]
    yl = _dot_t(h0, w2l[:, :H]) + _dot_t(h1, w2l[:, H:])
    y0_ref[...] = yl[:, :H]
    y1_ref[...] = yl[:, H:]
    z2_ref[...] = _dot_t(h0, w2r[:, :H]) + _dot_t(h1, w2r[:, H:]) + b2_ref[...]


def _mm2(a0, a1, z1, deg16, w2l, w2r, b2r):
    return pl.pallas_call(
        _mm2_body,
        grid=(GRID,),
        in_specs=[
            pl.BlockSpec((RB, H), lambda i: (i, 0)),
            pl.BlockSpec((RB, H), lambda i: (i, 0)),
            pl.BlockSpec((RB, D), lambda i: (i, 0)),
            pl.BlockSpec((RB, 1), lambda i: (i, 0)),
            pl.BlockSpec((D, D), lambda i: (0, 0)),
            pl.BlockSpec((D, D), lambda i: (0, 0)),
            pl.BlockSpec((1, D), lambda i: (0, 0)),
        ],
        out_specs=[
            pl.BlockSpec((RB, H), lambda i: (i, 0)),
            pl.BlockSpec((RB, H), lambda i: (i, 0)),
            pl.BlockSpec((RB, D), lambda i: (i, 0)),
        ],
        out_shape=[
            jax.ShapeDtypeStruct((N, H), _f32),
            jax.ShapeDtypeStruct((N, H), _f32),
            jax.ShapeDtypeStruct((N, D), _f32),
        ],
    )(a0, a1, z1, deg16, w2l, w2r, b2r)


def _mm3_body(a0_ref, a1_ref, z2_ref, deg_ref, out_ref):
    dinv = 1.0 / jnp.maximum(deg_ref[...], 1.0)
    z2 = z2_ref[...]
    out_ref[...] = jnp.concatenate(
        [a0_ref[...] * dinv + z2[:, :H], a1_ref[...] * dinv + z2[:, H:]],
        axis=1)


def _mm3(a0, a1, z2, deg16):
    return pl.pallas_call(
        _mm3_body,
        grid=(GRID,),
        in_specs=[
            pl.BlockSpec((RB, H), lambda i: (i, 0)),
            pl.BlockSpec((RB, H), lambda i: (i, 0)),
            pl.BlockSpec((RB, D), lambda i: (i, 0)),
            pl.BlockSpec((RB, 1), lambda i: (i, 0)),
        ],
        out_specs=pl.BlockSpec((RB, D), lambda i: (i, 0)),
        out_shape=jax.ShapeDtypeStruct((N, D), _f32),
    )(a0, a1, z2, deg16)


def _make_sc(with_deg: bool):
    mesh = plsc.VectorSubcoreMesh(core_axis_name="c", subcore_axis_name="s")
    out_type = [
        jax.ShapeDtypeStruct((N, H), _f32),
        jax.ShapeDtypeStruct((N, H), _f32),
    ]
    if with_deg:
        out_type.append(jax.ShapeDtypeStruct((N,), _f32))
    scratch = [
        pltpu.VMEM((CH,), jnp.int32),
        pltpu.VMEM((CH,), jnp.int32),
        pltpu.VMEM((CH, H), _f32),
        pltpu.VMEM((ROWS_Z + 8,), _f32),
        pltpu.VMEM((CH,), _f32),
        pltpu.VMEM_SHARED((N_PAD, H), _f32),
        pltpu.VMEM_SHARED((N_PAD,), _f32),
        pltpu.SemaphoreType.DMA,
    ]

    def body(y0_hbm, y1_hbm, src_hbm, dst_hbm, agg0_hbm, agg1_hbm, *rest):
        if with_deg:
            deg_hbm = rest[0]
            idx_s, idx_d, rows, dbnc, ones_v, acc, dacc, sem = rest[1:]
        else:
            idx_s, idx_d, rows, dbnc, ones_v, acc, dacc, sem = rest

        c = lax.axis_index("c")
        s = lax.axis_index("s")

        def zrow(r, carry):
            for j in range(H // 16):
                rows[r, pl.ds(j * 16, 16)] = jnp.zeros((16,), _f32)
            return carry

        lax.fori_loop(0, CH, zrow, 0)

        def zflat(r, carry):
            dbnc[pl.ds(r * 16, 16)] = jnp.zeros((16,), _f32)
            return carry

        lax.fori_loop(0, (ROWS_Z + 8) // 16, zflat, 0)

        z0 = s * ROWS_Z
        for k in range(ROWS_Z // CH):
            pltpu.sync_copy(rows, acc.at[pl.ds(z0 + k * CH, CH), :])
        zr = ROWS_Z % CH
        pltpu.sync_copy(rows.at[pl.ds(0, zr), :],
                        acc.at[pl.ds(z0 + (ROWS_Z // CH) * CH, zr), :])
        if with_deg:
            pltpu.sync_copy(dbnc.at[pl.ds(0, ROWS_Z)],
                            dacc.at[pl.ds(z0, ROWS_Z)])

            def onerow(r, carry):
                ones_v[pl.ds(r * 16, 16)] = jnp.full((16,), 1.0, _f32)
                return carry

            lax.fori_loop(0, CH // 16, onerow, 0)
        plsc.subcore_barrier()

        ebase = s * (E_PAD // 16)

        def chunk(i, carry):
            base = ebase + i * CH
            pltpu.sync_copy(src_hbm.at[pl.ds(base, CH)], idx_s)
            pltpu.sync_copy(dst_hbm.at[pl.ds(base, CH)], idx_d)

            @pl.when(c == 0)
            def _():
                pltpu.async_copy(y0_hbm.at[idx_s], rows, sem).wait()

            @pl.when(c == 1)
            def _():
                pltpu.async_copy(y1_hbm.at[idx_s], rows, sem).wait()

            pltpu.sync_copy(rows, acc.at[idx_d], add=True)
            if with_deg:
                @pl.when(c == 1)
                def _():
                    pltpu.sync_copy(ones_v, dacc.at[idx_d], add=True)
            return carry

        lax.fori_loop(0, CHUNKS, chunk, 0)
        plsc.subcore_barrier()

        w0 = s * ROWS_W

        def wb_chunk(off, rows_n):
            pltpu.sync_copy(acc.at[pl.ds(w0 + off, rows_n), :],
                            rows.at[pl.ds(0, rows_n), :])

            @pl.when(c == 0)
            def _():
                pltpu.sync_copy(rows.at[pl.ds(0, rows_n), :],
                                agg0_hbm.at[pl.ds(w0 + off, rows_n), :])

            @pl.when(c == 1)
            def _():
                pltpu.sync_copy(rows.at[pl.ds(0, rows_n), :],
                                agg1_hbm.at[pl.ds(w0 + off, rows_n), :])

        for k in range(4):
            wb_chunk(k * CH, CH)

        @pl.when(s < 15)
        def _():
            wb_chunk(4 * CH, ROWS_W - 4 * CH)

        @pl.when(s == 15)
        def _():
            wb_chunk(4 * CH, ROWS_W_LAST - 4 * CH)

        if with_deg:
            @pl.when(c == 1)
            def _():
                def wb_deg(rows_n):
                    pltpu.sync_copy(dacc.at[pl.ds(w0, rows_n)],
                                    dbnc.at[pl.ds(0, rows_n)])
                    pltpu.sync_copy(dbnc.at[pl.ds(0, rows_n)],
                                    deg_hbm.at[pl.ds(w0, rows_n)])

                @pl.when(s < 15)
                def _():
                    wb_deg(ROWS_W)

                @pl.when(s == 15)
                def _():
                    wb_deg(ROWS_W_LAST)

    return pl.kernel(body, out_type=out_type, mesh=mesh,
                     scratch_types=scratch)


_sc_deg = _make_sc(with_deg=True)
_sc_plain = _make_sc(with_deg=False)


@jax.jit
def kernel(x, edge_index, W1l, W1r, b1, W2l, W2r, b2):
    src = edge_index[0].astype(jnp.int32)
    dst = edge_index[1].astype(jnp.int32)
    pad = E_PAD - E
    src = jnp.concatenate([src, jnp.zeros((pad,), jnp.int32)])
    dst = jnp.concatenate([dst, jnp.full((pad,), N, jnp.int32)])
    b1r = b1.reshape(1, D)
    b2r = b2.reshape(1, D)

    y0, y1, z1 = _mm1(x, W1l, W1r, b1r)
    a0, a1, deg = _sc_deg(y0, y1, src, dst)
    deg16 = deg.reshape(N, 1)
    y20, y21, z2 = _mm2(a0, a1, z1, deg16, W2l, W2r, b2r)
    a20, a21 = _sc_plain(y20, y21, src, dst)
    return _mm3(a20, a21, z2, deg16)


# speedup vs baseline: 3.7303x; 1.3927x over previous
import functools

import jax
import jax.numpy as jnp
from jax import lax
from jax.experimental import pallas as pl
from jax.experimental.pallas import tpu as pltpu
from jax.experimental.pallas import tpu_sc as plsc

N = 10000
E = 160000
D = 256
H = 128
N_PAD = 10112
E_PAD = 163840
CH = 128
CHUNKS = E_PAD // (16 * CH)
ROWS_Z = N_PAD // 16
ROWS_W = 632
ROWS_W_LAST = N - 15 * ROWS_W
RB = 400
GRID = N // RB

_f32 = jnp.float32


def _dot_t(a, w):
    return lax.dot_general(a, w, (((1,), (1,)), ((), ())),
                           preferred_element_type=_f32)


def _mm1_body(x_ref, w1l_ref, w1r_ref, b1_ref, y0_ref, y1_ref, z1_ref):
    xb = x_ref[...]
    yl = _dot_t(xb, w1l_ref[...])
    y0_ref[...] = yl[:, :H]
    y1_ref[...] = yl[:, H:]
    z1_ref[...] = _dot_t(xb, w1r_ref[...]) + b1_ref[...]


def _mm1(x, w1l, w1r, b1r):
    return pl.pallas_call(
        _mm1_body,
        grid=(GRID,),
        in_specs=[
            pl.BlockSpec((RB, D), lambda i: (i, 0)),
            pl.BlockSpec((D, D), lambda i: (0, 0)),
            pl.BlockSpec((D, D), lambda i: (0, 0)),
            pl.BlockSpec((1, D), lambda i: (0, 0)),
        ],
        out_specs=[
            pl.BlockSpec((RB, H), lambda i: (i, 0)),
            pl.BlockSpec((RB, H), lambda i: (i, 0)),
            pl.BlockSpec((RB, D), lambda i: (i, 0)),
        ],
        out_shape=[
            jax.ShapeDtypeStruct((N, H), _f32),
            jax.ShapeDtypeStruct((N, H), _f32),
            jax.ShapeDtypeStruct((N, D), _f32),
        ],
    )(x, w1l, w1r, b1r)


def _mm2_body(a0_ref, a1_ref, z1_ref, deg_ref, w2l_ref, w2r_ref, b2_ref,
              y0_ref, y1_ref, z2_ref):
    dinv = 1.0 / jnp.maximum(deg_ref[...], 1.0)
    z1 = z1_ref[...]
    h0 = jnp.maximum(a0_ref[...] * dinv + z1[:, :H], 0.0)
    h1 = jnp.maximum(a1_ref[...] * dinv + z1[:, H:], 0.0)
    w2l = w2l_ref[...]
    w2r = w2r_ref[...]
    yl = _dot_t(h0, w2l[:, :H]) + _dot_t(h1, w2l[:, H:])
    y0_ref[...] = yl[:, :H]
    y1_ref[...] = yl[:, H:]
    z2_ref[...] = _dot_t(h0, w2r[:, :H]) + _dot_t(h1, w2r[:, H:]) + b2_ref[...]


def _mm2(a0, a1, z1, deg16, w2l, w2r, b2r):
    return pl.pallas_call(
        _mm2_body,
        grid=(GRID,),
        in_specs=[
            pl.BlockSpec((RB, H), lambda i: (i, 0)),
            pl.BlockSpec((RB, H), lambda i: (i, 0)),
            pl.BlockSpec((RB, D), lambda i: (i, 0)),
            pl.BlockSpec((RB, 1), lambda i: (i, 0)),
            pl.BlockSpec((D, D), lambda i: (0, 0)),
            pl.BlockSpec((D, D), lambda i: (0, 0)),
            pl.BlockSpec((1, D), lambda i: (0, 0)),
        ],
        out_specs=[
            pl.BlockSpec((RB, H), lambda i: (i, 0)),
            pl.BlockSpec((RB, H), lambda i: (i, 0)),
            pl.BlockSpec((RB, D), lambda i: (i, 0)),
        ],
        out_shape=[
            jax.ShapeDtypeStruct((N, H), _f32),
            jax.ShapeDtypeStruct((N, H), _f32),
            jax.ShapeDtypeStruct((N, D), _f32),
        ],
    )(a0, a1, z1, deg16, w2l, w2r, b2r)


def _mm3_body(a0_ref, a1_ref, z2_ref, deg_ref, out_ref):
    dinv = 1.0 / jnp.maximum(deg_ref[...], 1.0)
    z2 = z2_ref[...]
    out_ref[...] = jnp.concatenate(
        [a0_ref[...] * dinv + z2[:, :H], a1_ref[...] * dinv + z2[:, H:]],
        axis=1)


def _mm3(a0, a1, z2, deg16):
    return pl.pallas_call(
        _mm3_body,
        grid=(GRID,),
        in_specs=[
            pl.BlockSpec((RB, H), lambda i: (i, 0)),
            pl.BlockSpec((RB, H), lambda i: (i, 0)),
            pl.BlockSpec((RB, D), lambda i: (i, 0)),
            pl.BlockSpec((RB, 1), lambda i: (i, 0)),
        ],
        out_specs=pl.BlockSpec((RB, D), lambda i: (i, 0)),
        out_shape=jax.ShapeDtypeStruct((N, D), _f32),
    )(a0, a1, z2, deg16)


def _make_sc(with_deg: bool):
    mesh = plsc.VectorSubcoreMesh(core_axis_name="c", subcore_axis_name="s")
    out_type = [
        jax.ShapeDtypeStruct((N, H), _f32),
        jax.ShapeDtypeStruct((N, H), _f32),
    ]
    if with_deg:
        out_type.append(jax.ShapeDtypeStruct((N,), _f32))
    scratch = [
        pltpu.VMEM((CH,), jnp.int32),
        pltpu.VMEM((CH,), jnp.int32),
        pltpu.VMEM((CH,), jnp.int32),
        pltpu.VMEM((CH,), jnp.int32),
        pltpu.VMEM((CH, H), _f32),
        pltpu.VMEM((CH, H), _f32),
        pltpu.VMEM((ROWS_Z + 8,), _f32),
        pltpu.VMEM((CH,), _f32),
        pltpu.VMEM_SHARED((N_PAD, H), _f32),
        pltpu.VMEM_SHARED((N_PAD,), _f32),
        pltpu.SemaphoreType.DMA,
        pltpu.SemaphoreType.DMA,
        pltpu.SemaphoreType.DMA,
        pltpu.SemaphoreType.DMA,
        pltpu.SemaphoreType.DMA,
        pltpu.SemaphoreType.DMA,
    ]

    def body(y0_hbm, y1_hbm, src_hbm, dst_hbm, agg0_hbm, agg1_hbm, *rest):
        if with_deg:
            deg_hbm = rest[0]
            rest = rest[1:]
        (idx_s0, idx_d0, idx_s1, idx_d1, rows0, rows1, dbnc, ones_v,
         acc, dacc, gsem0, gsem1, ssem0, ssem1, dsem0, dsem1) = rest
        rows = rows0
        idx_sl = ((idx_s0, idx_d0, rows0, gsem0, ssem0, dsem0),
                  (idx_s1, idx_d1, rows1, gsem1, ssem1, dsem1))

        c = lax.axis_index("c")
        s = lax.axis_index("s")

        def zrow(r, carry):
            for j in range(H // 16):
                rows[r, pl.ds(j * 16, 16)] = jnp.zeros((16,), _f32)
            return carry

        lax.fori_loop(0, CH, zrow, 0)

        def zflat(r, carry):
            dbnc[pl.ds(r * 16, 16)] = jnp.zeros((16,), _f32)
            return carry

        lax.fori_loop(0, (ROWS_Z + 8) // 16, zflat, 0)

        z0 = s * ROWS_Z
        for k in range(ROWS_Z // CH):
            pltpu.sync_copy(rows, acc.at[pl.ds(z0 + k * CH, CH), :])
        zr = ROWS_Z % CH
        pltpu.sync_copy(rows.at[pl.ds(0, zr), :],
                        acc.at[pl.ds(z0 + (ROWS_Z // CH) * CH, zr), :])
        if with_deg:
            pltpu.sync_copy(dbnc.at[pl.ds(0, ROWS_Z)],
                            dacc.at[pl.ds(z0, ROWS_Z)])

            def onerow(r, carry):
                ones_v[pl.ds(r * 16, 16)] = jnp.full((16,), 1.0, _f32)
                return carry

            lax.fori_loop(0, CH // 16, onerow, 0)
        plsc.subcore_barrier()

        ebase = s * (E_PAD // 16)

        def idx_load(k, i_s, i_d):
            base = ebase + k * CH
            pltpu.sync_copy(src_hbm.at[pl.ds(base, CH)], i_s)
            pltpu.sync_copy(dst_hbm.at[pl.ds(base, CH)], i_d)

        def gather_start(i_s, rbuf, gsem):
            @pl.when(c == 0)
            def _():
                pltpu.async_copy(y0_hbm.at[i_s], rbuf, gsem)

            @pl.when(c == 1)
            def _():
                pltpu.async_copy(y1_hbm.at[i_s], rbuf, gsem)

        def gather_wait(i_s, rbuf, gsem):
            pltpu.make_async_copy(y0_hbm.at[i_s], rbuf, gsem).wait()

        def scat_start(rbuf, i_d, ssem, dsem):
            pltpu.async_copy(rbuf, acc.at[i_d], ssem, add=True)
            if with_deg:
                @pl.when(c == 1)
                def _():
                    pltpu.async_copy(ones_v, dacc.at[i_d], dsem, add=True)

        def scat_wait(rbuf, i_d, ssem, dsem):
            pltpu.make_async_copy(rbuf, acc.at[i_d], ssem).wait()
            if with_deg:
                @pl.when(c == 1)
                def _():
                    pltpu.make_async_copy(ones_v, dacc.at[i_d], dsem).wait()

        s0 = idx_sl[0]
        s1 = idx_sl[1]

        idx_load(0, s0[0], s0[1])
        gather_start(s0[0], s0[2], s0[3])

        def pair(g, carry):
            c1 = 2 * g + 1

            @pl.when(g > 0)
            def _():
                scat_wait(s1[2], s1[1], s1[4], s1[5])

            idx_load(c1, s1[0], s1[1])
            gather_start(s1[0], s1[2], s1[3])

            gather_wait(s0[0], s0[2], s0[3])
            scat_start(s0[2], s0[1], s0[4], s0[5])

            scat_wait(s0[2], s0[1], s0[4], s0[5])
            idx_load(c1 + 1, s0[0], s0[1])
            gather_start(s0[0], s0[2], s0[3])

            gather_wait(s1[0], s1[2], s1[3])
            scat_start(s1[2], s1[1], s1[4], s1[5])
            return carry

        lax.fori_loop(0, CHUNKS // 2, pair, 0)
        gather_wait(s0[0], s0[2], s0[3])
        scat_wait(s1[2], s1[1], s1[4], s1[5])
        plsc.subcore_barrier()

        w0 = s * ROWS_W

        def wb_chunk(off, rows_n):
            pltpu.sync_copy(acc.at[pl.ds(w0 + off, rows_n), :],
                            rows.at[pl.ds(0, rows_n), :])

            @pl.when(c == 0)
            def _():
                pltpu.sync_copy(rows.at[pl.ds(0, rows_n), :],
                                agg0_hbm.at[pl.ds(w0 + off, rows_n), :])

            @pl.when(c == 1)
            def _():
                pltpu.sync_copy(rows.at[pl.ds(0, rows_n), :],
                                agg1_hbm.at[pl.ds(w0 + off, rows_n), :])

        for k in range(4):
            wb_chunk(k * CH, CH)

        @pl.when(s < 15)
        def _():
            wb_chunk(4 * CH, ROWS_W - 4 * CH)

        @pl.when(s == 15)
        def _():
            wb_chunk(4 * CH, ROWS_W_LAST - 4 * CH)

        if with_deg:
            @pl.when(c == 1)
            def _():
                def wb_deg(rows_n):
                    pltpu.sync_copy(dacc.at[pl.ds(w0, rows_n)],
                                    dbnc.at[pl.ds(0, rows_n)])
                    pltpu.sync_copy(dbnc.at[pl.ds(0, rows_n)],
                                    deg_hbm.at[pl.ds(w0, rows_n)])

                @pl.when(s < 15)
                def _():
                    wb_deg(ROWS_W)

                @pl.when(s == 15)
                def _():
                    wb_deg(ROWS_W_LAST)

    return pl.kernel(body, out_type=out_type, mesh=mesh,
                     scratch_types=scratch)


_sc_deg = _make_sc(with_deg=True)
_sc_plain = _make_sc(with_deg=False)


@jax.jit
def kernel(x, edge_index, W1l, W1r, b1, W2l, W2r, b2):
    src = edge_index[0].astype(jnp.int32)
    dst = edge_index[1].astype(jnp.int32)
    pad = E_PAD + CH - E
    src = jnp.concatenate([src, jnp.zeros((pad,), jnp.int32)])
    dst = jnp.concatenate([dst, jnp.full((pad,), N, jnp.int32)])
    b1r = b1.reshape(1, D)
    b2r = b2.reshape(1, D)

    y0, y1, z1 = _mm1(x, W1l, W1r, b1r)
    a0, a1, deg = _sc_deg(y0, y1, src, dst)
    deg16 = deg.reshape(N, 1)
    y20, y21, z2 = _mm2(a0, a1, z1, deg16, W2l, W2r, b2r)
    a20, a21 = _sc_plain(y20, y21, src, dst)
    return _mm3(a20, a21, z2, deg16)


# speedup vs baseline: 3.9612x; 1.0619x over previous
import functools

import jax
import jax.numpy as jnp
from jax import lax
from jax.experimental import pallas as pl
from jax.experimental.pallas import tpu as pltpu
from jax.experimental.pallas import tpu_sc as plsc

N = 10000
E = 160000
D = 256
H = 128
N_PAD = 10112
E_PAD = 163840
CH = 128
CHUNKS = E_PAD // (16 * CH)
ROWS_Z = N_PAD // 16
ROWS_W = 632
ROWS_W_LAST = N - 15 * ROWS_W
HN = CHUNKS // 2
RB = 400
GRID = N // RB

_f32 = jnp.float32


def _dot_t(a, w):
    return lax.dot_general(a, w, (((1,), (1,)), ((), ())),
                           preferred_element_type=_f32)


def _mm1_body(x_ref, w1l_ref, w1r_ref, b1_ref, y0_ref, y1_ref, z1_ref):
    xb = x_ref[...]
    yl = _dot_t(xb, w1l_ref[...])
    y0_ref[...] = yl[:, :H]
    y1_ref[...] = yl[:, H:]
    z1_ref[...] = _dot_t(xb, w1r_ref[...]) + b1_ref[...]


def _mm1(x, w1l, w1r, b1r):
    return pl.pallas_call(
        _mm1_body,
        grid=(GRID,),
        in_specs=[
            pl.BlockSpec((RB, D), lambda i: (i, 0)),
            pl.BlockSpec((D, D), lambda i: (0, 0)),
            pl.BlockSpec((D, D), lambda i: (0, 0)),
            pl.BlockSpec((1, D), lambda i: (0, 0)),
        ],
        out_specs=[
            pl.BlockSpec((RB, H), lambda i: (i, 0)),
            pl.BlockSpec((RB, H), lambda i: (i, 0)),
            pl.BlockSpec((RB, D), lambda i: (i, 0)),
        ],
        out_shape=[
            jax.ShapeDtypeStruct((N, H), _f32),
            jax.ShapeDtypeStruct((N, H), _f32),
            jax.ShapeDtypeStruct((N, D), _f32),
        ],
    )(x, w1l, w1r, b1r)


def _mm2_body(a0_ref, a1_ref, z1_ref, deg_ref, w2l_ref, w2r_ref, b2_ref,
              y0_ref, y1_ref, z2_ref):
    dinv = 1.0 / jnp.maximum(deg_ref[...], 1.0)
    z1 = z1_ref[...]
    h0 = jnp.maximum(a0_ref[...] * dinv + z1[:, :H], 0.0)
    h1 = jnp.maximum(a1_ref[...] * dinv + z1[:, H:], 0.0)
    w2l = w2l_ref[...]
    w2r = w2r_ref[...]
    yl = _dot_t(h0, w2l[:, :H]) + _dot_t(h1, w2l[:, H:])
    y0_ref[...] = yl[:, :H]
    y1_ref[...] = yl[:, H:]
    z2_ref[...] = _dot_t(h0, w2r[:, :H]) + _dot_t(h1, w2r[:, H:]) + b2_ref[...]


def _mm2(a0, a1, z1, deg16, w2l, w2r, b2r):
    return pl.pallas_call(
        _mm2_body,
        grid=(GRID,),
        in_specs=[
            pl.BlockSpec((RB, H), lambda i: (i, 0)),
            pl.BlockSpec((RB, H), lambda i: (i, 0)),
            pl.BlockSpec((RB, D), lambda i: (i, 0)),
            pl.BlockSpec((RB, 1), lambda i: (i, 0)),
            pl.BlockSpec((D, D), lambda i: (0, 0)),
            pl.BlockSpec((D, D), lambda i: (0, 0)),
            pl.BlockSpec((1, D), lambda i: (0, 0)),
        ],
        out_specs=[
            pl.BlockSpec((RB, H), lambda i: (i, 0)),
            pl.BlockSpec((RB, H), lambda i: (i, 0)),
            pl.BlockSpec((RB, D), lambda i: (i, 0)),
        ],
        out_shape=[
            jax.ShapeDtypeStruct((N, H), _f32),
            jax.ShapeDtypeStruct((N, H), _f32),
            jax.ShapeDtypeStruct((N, D), _f32),
        ],
    )(a0, a1, z1, deg16, w2l, w2r, b2r)


def _mm3_body(a0_ref, a1_ref, z2_ref, deg_ref, out_ref):
    dinv = 1.0 / jnp.maximum(deg_ref[...], 1.0)
    z2 = z2_ref[...]
    out_ref[...] = jnp.concatenate(
        [a0_ref[...] * dinv + z2[:, :H], a1_ref[...] * dinv + z2[:, H:]],
        axis=1)


def _mm3(a0, a1, z2, deg16):
    return pl.pallas_call(
        _mm3_body,
        grid=(GRID,),
        in_specs=[
            pl.BlockSpec((RB, H), lambda i: (i, 0)),
            pl.BlockSpec((RB, H), lambda i: (i, 0)),
            pl.BlockSpec((RB, D), lambda i: (i, 0)),
            pl.BlockSpec((RB, 1), lambda i: (i, 0)),
        ],
        out_specs=pl.BlockSpec((RB, D), lambda i: (i, 0)),
        out_shape=jax.ShapeDtypeStruct((N, D), _f32),
    )(a0, a1, z2, deg16)


def _make_sc(with_deg: bool):
    mesh = plsc.VectorSubcoreMesh(core_axis_name="c", subcore_axis_name="s")
    out_type = [
        jax.ShapeDtypeStruct((N, H), _f32),
        jax.ShapeDtypeStruct((N, H), _f32),
    ]
    if with_deg:
        out_type.append(jax.ShapeDtypeStruct((N,), _f32))
    scratch = [
        pltpu.VMEM((HN + 8, CH), jnp.int32),
        pltpu.VMEM((HN, CH), jnp.int32),
        pltpu.VMEM((CH, H), _f32),
        pltpu.VMEM((CH, H), _f32),
        pltpu.VMEM((ROWS_Z + 8,), _f32),
        pltpu.VMEM((CH,), _f32),
        pltpu.VMEM_SHARED((N_PAD, H), _f32),
        pltpu.VMEM_SHARED((N_PAD,), _f32),
        pltpu.SemaphoreType.DMA,
        pltpu.SemaphoreType.DMA,
        pltpu.SemaphoreType.DMA,
        pltpu.SemaphoreType.DMA,
        pltpu.SemaphoreType.DMA,
        pltpu.SemaphoreType.DMA,
    ]

    def body(y0_hbm, y1_hbm, src_hbm, dst_hbm, agg0_hbm, agg1_hbm, *rest):
        if with_deg:
            deg_hbm = rest[0]
            rest = rest[1:]
        (src_all, dst_all, rows0, rows1, dbnc, ones_v,
         acc, dacc, gsem0, gsem1, ssem0, ssem1, dsem0, dsem1) = rest
        rows = rows0

        c = lax.axis_index("c")
        s = lax.axis_index("s")

        def zrow(r, carry):
            for j in range(H // 16):
                rows[r, pl.ds(j * 16, 16)] = jnp.zeros((16,), _f32)
            return carry

        lax.fori_loop(0, CH, zrow, 0)

        def zflat(r, carry):
            dbnc[pl.ds(r * 16, 16)] = jnp.zeros((16,), _f32)
            return carry

        lax.fori_loop(0, (ROWS_Z + 8) // 16, zflat, 0)

        z0 = s * ROWS_Z
        for k in range(ROWS_Z // CH):
            pltpu.sync_copy(rows, acc.at[pl.ds(z0 + k * CH, CH), :])
        zr = ROWS_Z % CH
        pltpu.sync_copy(rows.at[pl.ds(0, zr), :],
                        acc.at[pl.ds(z0 + (ROWS_Z // CH) * CH, zr), :])
        if with_deg:
            pltpu.sync_copy(dbnc.at[pl.ds(0, ROWS_Z)],
                            dacc.at[pl.ds(z0, ROWS_Z)])

            def onerow(r, carry):
                ones_v[pl.ds(r * 16, 16)] = jnp.full((16,), 1.0, _f32)
                return carry

            lax.fori_loop(0, CH // 16, onerow, 0)
        plsc.subcore_barrier()

        def gather_start(i_s, rbuf, gsem):
            @pl.when(c == 0)
            def _():
                pltpu.async_copy(y0_hbm.at[i_s], rbuf, gsem)

            @pl.when(c == 1)
            def _():
                pltpu.async_copy(y1_hbm.at[i_s], rbuf, gsem)

        def gather_wait(i_s, rbuf, gsem):
            pltpu.make_async_copy(y0_hbm.at[i_s], rbuf, gsem).wait()

        def scat_start(rbuf, i_d, ssem, dsem):
            pltpu.async_copy(rbuf, acc.at[i_d], ssem, add=True)
            if with_deg:
                @pl.when(c == 1)
                def _():
                    pltpu.async_copy(ones_v, dacc.at[i_d], dsem, add=True)

        def scat_wait(rbuf, i_d, ssem, dsem):
            pltpu.make_async_copy(rbuf, acc.at[i_d], ssem).wait()
            if with_deg:
                @pl.when(c == 1)
                def _():
                    pltpu.make_async_copy(ones_v, dacc.at[i_d], dsem).wait()

        def pair(g, carry):
            l1 = 2 * g + 1

            @pl.when(g > 0)
            def _():
                scat_wait(rows1, dst_all.at[0], ssem1, dsem1)

            gather_start(src_all.at[l1], rows1, gsem1)

            gather_wait(src_all.at[0], rows0, gsem0)
            scat_start(rows0, dst_all.at[2 * g], ssem0, dsem0)

            scat_wait(rows0, dst_all.at[0], ssem0, dsem0)
            gather_start(src_all.at[l1 + 1], rows0, gsem0)

            gather_wait(src_all.at[0], rows1, gsem1)
            scat_start(rows1, dst_all.at[l1], ssem1, dsem1)
            return carry

        for h in range(CHUNKS // HN):
            hbase = s * CHUNKS + h * HN
            pltpu.sync_copy(src_hbm.at[pl.ds(hbase, HN + 8), :], src_all)
            pltpu.sync_copy(dst_hbm.at[pl.ds(hbase, HN), :], dst_all)
            gather_start(src_all.at[0], rows0, gsem0)
            lax.fori_loop(0, HN // 2, pair, 0)
            gather_wait(src_all.at[0], rows0, gsem0)
            scat_wait(rows1, dst_all.at[0], ssem1, dsem1)
        plsc.subcore_barrier()

        w0 = s * ROWS_W

        def wb_chunk(off, rows_n):
            pltpu.sync_copy(acc.at[pl.ds(w0 + off, rows_n), :],
                            rows.at[pl.ds(0, rows_n), :])

            @pl.when(c == 0)
            def _():
                pltpu.sync_copy(rows.at[pl.ds(0, rows_n), :],
                                agg0_hbm.at[pl.ds(w0 + off, rows_n), :])

            @pl.when(c == 1)
            def _():
                pltpu.sync_copy(rows.at[pl.ds(0, rows_n), :],
                                agg1_hbm.at[pl.ds(w0 + off, rows_n), :])

        for k in range(4):
            wb_chunk(k * CH, CH)

        @pl.when(s < 15)
        def _():
            wb_chunk(4 * CH, ROWS_W - 4 * CH)

        @pl.when(s == 15)
        def _():
            wb_chunk(4 * CH, ROWS_W_LAST - 4 * CH)

        if with_deg:
            @pl.when(c == 1)
            def _():
                def wb_deg(rows_n):
                    pltpu.sync_copy(dacc.at[pl.ds(w0, rows_n)],
                                    dbnc.at[pl.ds(0, rows_n)])
                    pltpu.sync_copy(dbnc.at[pl.ds(0, rows_n)],
                                    deg_hbm.at[pl.ds(w0, rows_n)])

                @pl.when(s < 15)
                def _():
                    wb_deg(ROWS_W)

                @pl.when(s == 15)
                def _():
                    wb_deg(ROWS_W_LAST)

    return pl.kernel(body, out_type=out_type, mesh=mesh,
                     scratch_types=scratch)


_sc_deg = _make_sc(with_deg=True)
_sc_plain = _make_sc(with_deg=False)


@jax.jit
def kernel(x, edge_index, W1l, W1r, b1, W2l, W2r, b2):
    src = edge_index[0].astype(jnp.int32)
    dst = edge_index[1].astype(jnp.int32)
    pad = (16 * CHUNKS + 48) * CH - E
    src = jnp.concatenate([src, jnp.zeros((pad,), jnp.int32)]).reshape(-1, CH)
    dst = jnp.concatenate([dst, jnp.full((E_PAD - E,), N, jnp.int32)]).reshape(-1, CH)
    b1r = b1.reshape(1, D)
    b2r = b2.reshape(1, D)

    y0, y1, z1 = _mm1(x, W1l, W1r, b1r)
    a0, a1, deg = _sc_deg(y0, y1, src, dst)
    deg16 = deg.reshape(N, 1)
    y20, y21, z2 = _mm2(a0, a1, z1, deg16, W2l, W2r, b2r)
    a20, a21 = _sc_plain(y20, y21, src, dst)
    return _mm3(a20, a21, z2, deg16)
